# baseline (speedup 1.0000x reference)
import functools
from math import sqrt

import jax
import jax.numpy as jnp
from jax.experimental import pallas as pl
from jax.experimental.pallas import tpu as pltpu

B, L, DIM, H, D = 2, 2048, 2, 12, 64
U = 24
CH = 256
NCHUNK = L // CH
NTILE = 16
NPAIR = DIM * H // 2
NEG = -3.0e38
BIG = 4 * L
HIGHEST = jax.lax.Precision.HIGHEST


def _kernel_body(q_ref, k_ref, v_ref, lo_ref, hi_ref, out_ref,
                 s_ref, m_ref, sel_ref):
    f32 = jnp.float32
    col = jax.lax.broadcasted_iota(jnp.int32, (CH, 128), 1)
    col_valid = col < U

    def chunk_body(c, _):
        @pl.when(c < NCHUNK)
        def _matmul():
            buf = jax.lax.rem(c, 2)
            for sub in range(2):
                c0, c1 = sub * D, (sub + 1) * D
                qc = q_ref[pl.ds(c * CH, CH), c0:c1]
                s_ref[sub, buf, :, :] = jax.lax.dot_general(
                    qc, k_ref[:, c0:c1], (((1,), (1,)), ((), ())),
                    precision=HIGHEST, preferred_element_type=f32)

        @pl.when(c > 0)
        def _extract():
            cp = c - 1
            buf = jax.lax.rem(cp, 2)
            lo_c = lo_ref[pl.ds(cp * CH, CH), :]
            hi_c = hi_ref[pl.ds(cp * CH, CH), :]
            for sub in range(2):
                terms = []
                for t in range(NTILE):
                    gt = jnp.take_along_axis(
                        s_ref[sub, buf, :, t * 128:(t + 1) * 128],
                        lo_c, axis=1)
                    terms.append(jnp.where(hi_c == t, gt, 0.0))
                while len(terms) > 1:
                    terms = [a + b for a, b in zip(terms[::2], terms[1::2])]
                g = terms[0]
                gmax = jnp.max(jnp.where(col_valid, g, NEG), axis=1)
                gsum = jnp.sum(g, axis=1)
                m_ref[sub, cp, :] = gmax - gsum / float(L)
        return 0

    jax.lax.fori_loop(0, NCHUNK + 1, chunk_body, 0, unroll=False)

    flat = (jax.lax.broadcasted_iota(jnp.int32, (NCHUNK, CH), 0) * CH
            + jax.lax.broadcasted_iota(jnp.int32, (NCHUNK, CH), 1))

    def topk_body(t, carry):
        m0, m1 = carry
        outs = []
        for sub, m_val in enumerate((m0, m1)):
            mx = jnp.max(m_val)
            i = jnp.min(jnp.where(m_val == mx, flat, BIG))
            sel_ref[sub, t] = i
            outs.append(jnp.where(flat == i, NEG, m_val))
        return tuple(outs)

    jax.lax.fori_loop(0, U, topk_body, (m_ref[0, :, :], m_ref[1, :, :]),
                      unroll=False)

    for sub in range(2):
        c0, c1 = sub * D, (sub + 1) * D
        V_val = v_ref[:, c0:c1]
        rows = []
        for t in range(U):
            rows.append(q_ref[pl.ds(sel_ref[sub, t], 1), c0:c1])
        q_sel = jnp.concatenate(rows, axis=0)
        scores = jax.lax.dot_general(
            q_sel, k_ref[:, c0:c1], (((1,), (1,)), ((), ())),
            precision=HIGHEST, preferred_element_type=f32) * (1.0 / sqrt(D))
        smax = jnp.max(scores, axis=1, keepdims=True)
        unnorm = jnp.exp(scores - smax)
        attn = unnorm / jnp.sum(unnorm, axis=1, keepdims=True)
        out24 = jax.lax.dot_general(
            attn, V_val, (((1,), (0,)), ((), ())),
            preferred_element_type=f32)
        vmean = jnp.mean(V_val, axis=0, keepdims=True)
        out_ref[sub, :, :] = jnp.broadcast_to(vmean, (L, D))
        for t in range(U):
            out_ref[sub, pl.ds(sel_ref[sub, t], 1), :] = out24[t:t + 1, :]


@jax.jit
def kernel(queries, keys, values, attn_mask):
    del attn_mask
    idx = jax.random.randint(jax.random.key(42), (L, U), 0, L)
    lo = jnp.concatenate(
        [idx % 128, jnp.zeros((L, 128 - U), jnp.int32)], axis=1)
    hi = jnp.concatenate(
        [idx // 128, jnp.full((L, 128 - U), -1, jnp.int32)], axis=1)

    qkv_spec = pl.BlockSpec(
        (L, 128), lambda sp: (sp // NPAIR, sp % NPAIR))
    idx_spec = pl.BlockSpec((L, 128), lambda sp: (0, 0))
    out_spec = pl.BlockSpec((2, L, D), lambda sp: (sp, 0, 0))

    def flat(x):
        return x.reshape(B * L, DIM * H * D)

    out = pl.pallas_call(
        _kernel_body,
        grid=(B * NPAIR,),
        in_specs=[qkv_spec, qkv_spec, qkv_spec, idx_spec, idx_spec],
        out_specs=out_spec,
        out_shape=jax.ShapeDtypeStruct((B * DIM * H, L, D), jnp.float32),
        scratch_shapes=[
            pltpu.VMEM((2, 2, CH, L), jnp.float32),
            pltpu.VMEM((2, NCHUNK, CH), jnp.float32),
            pltpu.SMEM((2, U), jnp.int32),
        ],
    )(flat(queries), flat(keys), flat(values), lo, hi)
    return out.reshape(B, DIM, H, L, D)


# speedup vs baseline: 1.4076x; 1.4076x over previous
import functools
from math import sqrt

import jax
import jax.numpy as jnp
from jax.experimental import pallas as pl
from jax.experimental.pallas import tpu as pltpu

B, L, DIM, H, D = 2, 2048, 2, 12, 64
U = 24
NCHUNK = 16
NPAIR = DIM * H // 2
NEG = -3.0e38
BIG = 4 * L
HIGHEST = jax.lax.Precision.HIGHEST


def _kernel_body(q_ref, k_ref, v_ref, lo_ref, hi_ref, out_ref,
                 s_ref, m_ref, sel_ref):
    f32 = jnp.float32
    col = jax.lax.broadcasted_iota(jnp.int32, (128, 128), 1)
    col_valid = col < U

    def chunk_body(i, _):
        for half in range(2):
            c = i * 2 + half
            lo_c = lo_ref[pl.ds(c * 128, 128), :]
            hi_c = hi_ref[pl.ds(c * 128, 128), :]
            for sub in range(2):
                c0, c1 = sub * D, (sub + 1) * D
                qc = q_ref[pl.ds(c * 128, 128), c0:c1]
                s_ref[sub, half, :, :] = jax.lax.dot_general(
                    qc, k_ref[:, c0:c1], (((1,), (1,)), ((), ())),
                    precision=HIGHEST, preferred_element_type=f32)
                terms = []
                for t in range(NCHUNK):
                    gt = jnp.take_along_axis(
                        s_ref[sub, half, :, t * 128:(t + 1) * 128],
                        lo_c, axis=1)
                    terms.append(jnp.where(hi_c == t, gt, 0.0))
                while len(terms) > 1:
                    terms = [a + b for a, b in zip(terms[::2], terms[1::2])]
                g = terms[0]
                gmax = jnp.max(jnp.where(col_valid, g, NEG), axis=1)
                gsum = jnp.sum(g, axis=1)
                m_ref[sub, c, :] = gmax - gsum / float(L)
        return 0

    jax.lax.fori_loop(0, NCHUNK // 2, chunk_body, 0, unroll=False)

    flat = (jax.lax.broadcasted_iota(jnp.int32, (NCHUNK, 128), 0) * 128
            + jax.lax.broadcasted_iota(jnp.int32, (NCHUNK, 128), 1))

    def topk_body(t, carry):
        m0, m1 = carry
        outs = []
        for sub, m_val in enumerate((m0, m1)):
            mx = jnp.max(m_val)
            i = jnp.min(jnp.where(m_val == mx, flat, BIG))
            sel_ref[sub, t] = i
            outs.append(jnp.where(flat == i, NEG, m_val))
        return tuple(outs)

    jax.lax.fori_loop(0, U, topk_body, (m_ref[0, :, :], m_ref[1, :, :]),
                      unroll=False)

    for sub in range(2):
        c0, c1 = sub * D, (sub + 1) * D
        V_val = v_ref[:, c0:c1]
        rows = []
        for t in range(U):
            rows.append(q_ref[pl.ds(sel_ref[sub, t], 1), c0:c1])
        q_sel = jnp.concatenate(rows, axis=0)
        scores = jax.lax.dot_general(
            q_sel, k_ref[:, c0:c1], (((1,), (1,)), ((), ())),
            precision=HIGHEST, preferred_element_type=f32) * (1.0 / sqrt(D))
        smax = jnp.max(scores, axis=1, keepdims=True)
        unnorm = jnp.exp(scores - smax)
        attn = unnorm / jnp.sum(unnorm, axis=1, keepdims=True)
        out24 = jax.lax.dot_general(
            attn, V_val, (((1,), (0,)), ((), ())),
            preferred_element_type=f32)
        vmean = jnp.mean(V_val, axis=0, keepdims=True)
        out_ref[sub, :, :] = jnp.broadcast_to(vmean, (L, D))
        for t in range(U):
            out_ref[sub, pl.ds(sel_ref[sub, t], 1), :] = out24[t:t + 1, :]


@jax.jit
def kernel(queries, keys, values, attn_mask):
    del attn_mask
    idx = jax.random.randint(jax.random.key(42), (L, U), 0, L)
    lo = jnp.concatenate(
        [idx % 128, jnp.zeros((L, 128 - U), jnp.int32)], axis=1)
    hi = jnp.concatenate(
        [idx // 128, jnp.full((L, 128 - U), -1, jnp.int32)], axis=1)

    qkv_spec = pl.BlockSpec(
        (L, 128), lambda sp: (sp // NPAIR, sp % NPAIR))
    idx_spec = pl.BlockSpec((L, 128), lambda sp: (0, 0))
    out_spec = pl.BlockSpec((2, L, D), lambda sp: (sp, 0, 0))

    def flat(x):
        return x.reshape(B * L, DIM * H * D)

    out = pl.pallas_call(
        _kernel_body,
        grid=(B * NPAIR,),
        in_specs=[qkv_spec, qkv_spec, qkv_spec, idx_spec, idx_spec],
        out_specs=out_spec,
        out_shape=jax.ShapeDtypeStruct((B * DIM * H, L, D), jnp.float32),
        scratch_shapes=[
            pltpu.VMEM((2, 2, 128, L), jnp.float32),
            pltpu.VMEM((2, NCHUNK, 128), jnp.float32),
            pltpu.SMEM((2, U), jnp.int32),
        ],
    )(flat(queries), flat(keys), flat(values), lo, hi)
    return out.reshape(B, DIM, H, L, D)


# speedup vs baseline: 1.4376x; 1.0213x over previous
import functools
from math import sqrt

import jax
import jax.numpy as jnp
from jax.experimental import pallas as pl
from jax.experimental.pallas import tpu as pltpu

B, L, DIM, H, D = 2, 2048, 2, 12, 64
U = 24
NCHUNK = 16
NPAIR = DIM * H // 2
NEG = -3.0e38
BIG = 4 * L
HIGHEST = jax.lax.Precision.HIGHEST


def _kernel_body(q_ref, k_ref, v_ref, lo_ref, hi_ref, out_ref,
                 s_ref, m_ref, sel_ref):
    f32 = jnp.float32
    col = jax.lax.broadcasted_iota(jnp.int32, (128, 128), 1)
    col_valid = col < U

    def chunk_body(i, _):
        for half in range(4):
            c = i * 4 + half
            lo_c = lo_ref[pl.ds(c * 128, 128), :]
            hi_c = hi_ref[pl.ds(c * 128, 128), :]
            for sub in range(2):
                c0, c1 = sub * D, (sub + 1) * D
                qc = q_ref[pl.ds(c * 128, 128), c0:c1]
                s_ref[sub, half, :, :] = jax.lax.dot_general(
                    qc, k_ref[:, c0:c1], (((1,), (1,)), ((), ())),
                    precision=HIGHEST, preferred_element_type=f32)
                terms = []
                for t in range(NCHUNK):
                    gt = jnp.take_along_axis(
                        s_ref[sub, half, :, t * 128:(t + 1) * 128],
                        lo_c, axis=1)
                    terms.append(jnp.where(hi_c == t, gt, 0.0))
                while len(terms) > 1:
                    terms = [a + b for a, b in zip(terms[::2], terms[1::2])]
                g = terms[0]
                gmax = jnp.max(jnp.where(col_valid, g, NEG), axis=1)
                gsum = jnp.sum(g, axis=1)
                m_ref[sub, c, :] = gmax - gsum / float(L)
        return 0

    jax.lax.fori_loop(0, NCHUNK // 4, chunk_body, 0, unroll=False)

    flat = (jax.lax.broadcasted_iota(jnp.int32, (NCHUNK, 128), 0) * 128
            + jax.lax.broadcasted_iota(jnp.int32, (NCHUNK, 128), 1))

    def topk_body(t, carry):
        m0, m1 = carry
        outs = []
        for sub, m_val in enumerate((m0, m1)):
            mx = jnp.max(m_val)
            i = jnp.min(jnp.where(m_val == mx, flat, BIG))
            sel_ref[sub, t] = i
            outs.append(jnp.where(flat == i, NEG, m_val))
        return tuple(outs)

    jax.lax.fori_loop(0, U, topk_body, (m_ref[0, :, :], m_ref[1, :, :]),
                      unroll=False)

    for sub in range(2):
        c0, c1 = sub * D, (sub + 1) * D
        V_val = v_ref[:, c0:c1]
        rows = []
        for t in range(U):
            rows.append(q_ref[pl.ds(sel_ref[sub, t], 1), c0:c1])
        q_sel = jnp.concatenate(rows, axis=0)
        scores = jax.lax.dot_general(
            q_sel, k_ref[:, c0:c1], (((1,), (1,)), ((), ())),
            precision=HIGHEST, preferred_element_type=f32) * (1.0 / sqrt(D))
        smax = jnp.max(scores, axis=1, keepdims=True)
        unnorm = jnp.exp(scores - smax)
        attn = unnorm / jnp.sum(unnorm, axis=1, keepdims=True)
        out24 = jax.lax.dot_general(
            attn, V_val, (((1,), (0,)), ((), ())),
            preferred_element_type=f32)
        vmean = jnp.mean(V_val, axis=0, keepdims=True)
        out_ref[sub, :, :] = jnp.broadcast_to(vmean, (L, D))
        for t in range(U):
            out_ref[sub, pl.ds(sel_ref[sub, t], 1), :] = out24[t:t + 1, :]


@jax.jit
def kernel(queries, keys, values, attn_mask):
    del attn_mask
    idx = jax.random.randint(jax.random.key(42), (L, U), 0, L)
    lo = jnp.concatenate(
        [idx % 128, jnp.zeros((L, 128 - U), jnp.int32)], axis=1)
    hi = jnp.concatenate(
        [idx // 128, jnp.full((L, 128 - U), -1, jnp.int32)], axis=1)

    qkv_spec = pl.BlockSpec(
        (L, 128), lambda sp: (sp // NPAIR, sp % NPAIR))
    idx_spec = pl.BlockSpec((L, 128), lambda sp: (0, 0))
    out_spec = pl.BlockSpec((2, L, D), lambda sp: (sp, 0, 0))

    def flat(x):
        return x.reshape(B * L, DIM * H * D)

    out = pl.pallas_call(
        _kernel_body,
        grid=(B * NPAIR,),
        in_specs=[qkv_spec, qkv_spec, qkv_spec, idx_spec, idx_spec],
        out_specs=out_spec,
        out_shape=jax.ShapeDtypeStruct((B * DIM * H, L, D), jnp.float32),
        scratch_shapes=[
            pltpu.VMEM((2, 4, 128, L), jnp.float32),
            pltpu.VMEM((2, NCHUNK, 128), jnp.float32),
            pltpu.SMEM((2, U), jnp.int32),
        ],
    )(flat(queries), flat(keys), flat(values), lo, hi)
    return out.reshape(B, DIM, H, L, D)


# speedup vs baseline: 1.4506x; 1.0091x over previous
import functools
from math import sqrt

import jax
import jax.numpy as jnp
from jax.experimental import pallas as pl
from jax.experimental.pallas import tpu as pltpu

B, L, DIM, H, D = 2, 2048, 2, 12, 64
U = 24
NCHUNK = 16
NPAIR = DIM * H // 2
NEG = -3.0e38
BIG = 4 * L
HIGHEST = jax.lax.Precision.HIGHEST


def _kernel_body(q_ref, k_ref, v_ref, lo_ref, hi_ref, out_ref,
                 s_ref, m_ref, sel_ref):
    f32 = jnp.float32
    col = jax.lax.broadcasted_iota(jnp.int32, (128, 128), 1)
    col_valid = col < U

    def chunk_body(i, _):
        for half in range(8):
            c = i * 8 + half
            lo_c = lo_ref[pl.ds(c * 128, 128), :]
            hi_c = hi_ref[pl.ds(c * 128, 128), :]
            for sub in range(2):
                c0, c1 = sub * D, (sub + 1) * D
                qc = q_ref[pl.ds(c * 128, 128), c0:c1]
                s_ref[sub, half, :, :] = jax.lax.dot_general(
                    qc, k_ref[:, c0:c1], (((1,), (1,)), ((), ())),
                    precision=HIGHEST, preferred_element_type=f32)
                terms = []
                for t in range(NCHUNK):
                    gt = jnp.take_along_axis(
                        s_ref[sub, half, :, t * 128:(t + 1) * 128],
                        lo_c, axis=1)
                    terms.append(jnp.where(hi_c == t, gt, 0.0))
                while len(terms) > 1:
                    terms = [a + b for a, b in zip(terms[::2], terms[1::2])]
                g = terms[0]
                gmax = jnp.max(jnp.where(col_valid, g, NEG), axis=1)
                gsum = jnp.sum(g, axis=1)
                m_ref[sub, c, :] = gmax - gsum / float(L)
        return 0

    jax.lax.fori_loop(0, NCHUNK // 8, chunk_body, 0, unroll=False)

    flat = (jax.lax.broadcasted_iota(jnp.int32, (NCHUNK, 128), 0) * 128
            + jax.lax.broadcasted_iota(jnp.int32, (NCHUNK, 128), 1))

    def topk_body(t, carry):
        m0, m1 = carry
        outs = []
        for sub, m_val in enumerate((m0, m1)):
            mx = jnp.max(m_val)
            i = jnp.min(jnp.where(m_val == mx, flat, BIG))
            sel_ref[sub, t] = i
            outs.append(jnp.where(flat == i, NEG, m_val))
        return tuple(outs)

    jax.lax.fori_loop(0, U, topk_body, (m_ref[0, :, :], m_ref[1, :, :]),
                      unroll=False)

    for sub in range(2):
        c0, c1 = sub * D, (sub + 1) * D
        V_val = v_ref[:, c0:c1]
        rows = []
        for t in range(U):
            rows.append(q_ref[pl.ds(sel_ref[sub, t], 1), c0:c1])
        q_sel = jnp.concatenate(rows, axis=0)
        scores = jax.lax.dot_general(
            q_sel, k_ref[:, c0:c1], (((1,), (1,)), ((), ())),
            precision=HIGHEST, preferred_element_type=f32) * (1.0 / sqrt(D))
        smax = jnp.max(scores, axis=1, keepdims=True)
        unnorm = jnp.exp(scores - smax)
        attn = unnorm / jnp.sum(unnorm, axis=1, keepdims=True)
        out24 = jax.lax.dot_general(
            attn, V_val, (((1,), (0,)), ((), ())),
            preferred_element_type=f32)
        vmean = jnp.mean(V_val, axis=0, keepdims=True)
        out_ref[sub, :, :] = jnp.broadcast_to(vmean, (L, D))
        for t in range(U):
            out_ref[sub, pl.ds(sel_ref[sub, t], 1), :] = out24[t:t + 1, :]


@jax.jit
def kernel(queries, keys, values, attn_mask):
    del attn_mask
    idx = jax.random.randint(jax.random.key(42), (L, U), 0, L)
    lo = jnp.concatenate(
        [idx % 128, jnp.zeros((L, 128 - U), jnp.int32)], axis=1)
    hi = jnp.concatenate(
        [idx // 128, jnp.full((L, 128 - U), -1, jnp.int32)], axis=1)

    qkv_spec = pl.BlockSpec(
        (L, 128), lambda sp: (sp // NPAIR, sp % NPAIR))
    idx_spec = pl.BlockSpec((L, 128), lambda sp: (0, 0))
    out_spec = pl.BlockSpec((2, L, D), lambda sp: (sp, 0, 0))

    def flat(x):
        return x.reshape(B * L, DIM * H * D)

    out = pl.pallas_call(
        _kernel_body,
        grid=(B * NPAIR,),
        in_specs=[qkv_spec, qkv_spec, qkv_spec, idx_spec, idx_spec],
        out_specs=out_spec,
        out_shape=jax.ShapeDtypeStruct((B * DIM * H, L, D), jnp.float32),
        scratch_shapes=[
            pltpu.VMEM((2, 8, 128, L), jnp.float32),
            pltpu.VMEM((2, NCHUNK, 128), jnp.float32),
            pltpu.SMEM((2, U), jnp.int32),
        ],
    )(flat(queries), flat(keys), flat(values), lo, hi)
    return out.reshape(B, DIM, H, L, D)
